# baseline (device time: 23471 ns/iter reference)
import contextlib
import os

import jax
import jax.numpy as jnp
from jax import lax
from jax.experimental import pallas as pl
from jax.experimental.pallas import tpu as pltpu


def _scope(name):
    if os.environ.get("KERNEL_SCOPES") == "1":
        return jax.named_scope(name)
    return contextlib.nullcontext()


N_DEV = 16
NP = 4
NS = 4


def kernel(x, w_mat):
    m, k_local = x.shape
    _, n = w_mat.shape
    chunk = m // N_DEV
    group_rows = NP * chunk
    hc = n // 2
    qc = hc // NS

    def body(x_ref, w_ref, out_ref, p_ref,
             cw_send, cw_recv, ccw_send, ccw_recv, p2_send, p2_recv,
             cw_ssem, cw_rsem, ccw_ssem, ccw_rsem, p2_ssem, p2_rsem):
        my = lax.axis_index("i")
        z = my // NP
        j = lax.rem(my, NP)
        plane_left = z * NP + lax.rem(j + NP - 1, NP)
        plane_right = z * NP + lax.rem(j + 1, NP)

        barrier_sem = pltpu.get_barrier_semaphore()
        for nbr in [plane_left, plane_right]:
            pl.semaphore_signal(
                barrier_sem, inc=1,
                device_id=(nbr,), device_id_type=pl.DeviceIdType.MESH,
            )

        with _scope("gemm"):
            xp = jnp.concatenate(
                [
                    x_ref[(NP * zz + g) * chunk:(NP * zz + g + 1) * chunk, :]
                    for g in range(NP)
                    for zz in range(NP)
                ]
            ).astype(jnp.bfloat16)
            wb = w_ref[...].astype(jnp.bfloat16)
            p_ref[...] = jnp.dot(
                xp, wb, preferred_element_type=jnp.float32
            ).astype(jnp.bfloat16)

        with _scope("barrier_wait"):
            pl.semaphore_wait(barrier_sem, 2)

        def pgroup(g, lo, width):
            return p_ref[pl.ds(g * group_rows, group_rows), lo:lo + width].astype(
                jnp.float32
            )

        ablate = os.environ.get("ABLATE", "")

        streams = [
            (cw_send, cw_recv, cw_ssem, cw_rsem, k * qc, plane_right, +1)
            for k in range(NS)
        ] + [
            (ccw_send, ccw_recv, ccw_ssem, ccw_rsem, hc + k * qc, plane_left, -1)
            for k in range(NS)
        ]
        live = {}
        for s in range(NP - 1) if ablate not in ("p2", "none") else []:
            g_cw = lax.rem(j + 2 * NP - 1 - s, NP)
            g_ccw = lax.rem(j + 1 + s, NP)
            with _scope(f"hop{s}"):
                for idx, (sbuf, rbuf, ssem, rsem, lo, target, sgn) in enumerate(streams):
                    k = idx % NS
                    g = g_cw if sgn > 0 else g_ccw
                    acc = pgroup(g, lo, qc)
                    if s > 0:
                        live[(idx, s - 1)].wait_recv()
                        acc = acc + rbuf[s - 1, k].astype(jnp.float32)
                    sbuf[s, k] = acc.astype(jnp.bfloat16)
                    rdma = pltpu.make_async_remote_copy(
                        src_ref=sbuf.at[s, k],
                        dst_ref=rbuf.at[s, k],
                        send_sem=ssem.at[s, k],
                        recv_sem=rsem.at[s, k],
                        device_id=(target,),
                        device_id_type=pl.DeviceIdType.MESH,
                    )
                    rdma.start()
                    live[(idx, s)] = rdma

        def reduced_block(b, lo_k):
            sbuf, rbuf, ssem, rsem, lo, target, sgn = streams[lo_k]
            k = lo_k % NS
            base = p_ref[
                pl.ds(j * group_rows + b * chunk, chunk), lo:lo + qc
            ].astype(jnp.float32)
            if ablate in ("p2", "none"):
                return base
            return (
                base
                + rbuf[NP - 2, k, pl.ds(b * chunk, chunk), :].astype(jnp.float32)
            )

        p2_rdmas = []
        own_pieces = []
        with _scope("p2_send"):
            for lo_k in range(2 * NS):
                if ablate not in ("p2", "none"):
                    live[(lo_k, NP - 2)].wait_recv()
                for r in (1, 2, 3) if ablate not in ("p1", "none") else []:
                    b = lax.rem(z + NP - r, NP)
                    p2_send[r - 1, lo_k] = reduced_block(b, lo_k).astype(
                        jnp.bfloat16
                    )
                    rdma = pltpu.make_async_remote_copy(
                        src_ref=p2_send.at[r - 1, lo_k],
                        dst_ref=p2_recv.at[r - 1, lo_k],
                        send_sem=p2_ssem.at[r - 1, lo_k],
                        recv_sem=p2_rsem.at[r - 1, lo_k],
                        device_id=(b * NP + j,),
                        device_id_type=pl.DeviceIdType.MESH,
                    )
                    rdma.start()
                    p2_rdmas.append(rdma)
                own_pieces.append(reduced_block(z, lo_k))

        with _scope("p2_recv"):
            for rdma in p2_rdmas:
                rdma.wait_recv()
        with _scope("out"):
            total = jnp.concatenate(own_pieces, axis=1)
            if ablate not in ("p1", "none"):
                for r in (1, 2, 3):
                    total = total + jnp.concatenate(
                        [p2_recv[r - 1, lo_k] for lo_k in range(2 * NS)], axis=1
                    ).astype(jnp.float32)
            out_ref[...] = jnp.maximum(total, 0.0)

            for rdma in live.values():
                rdma.wait_send()
            for rdma in p2_rdmas:
                rdma.wait_send()

    return pl.pallas_call(
        body,
        out_shape=jax.ShapeDtypeStruct((chunk, n), jnp.float32),
        in_specs=[
            pl.BlockSpec(memory_space=pltpu.VMEM),
            pl.BlockSpec(memory_space=pltpu.VMEM),
        ],
        out_specs=pl.BlockSpec(memory_space=pltpu.VMEM),
        scratch_shapes=[
            pltpu.VMEM((m, n), jnp.bfloat16),
            pltpu.VMEM((NP - 1, NS, group_rows, qc), jnp.bfloat16),
            pltpu.VMEM((NP - 1, NS, group_rows, qc), jnp.bfloat16),
            pltpu.VMEM((NP - 1, NS, group_rows, qc), jnp.bfloat16),
            pltpu.VMEM((NP - 1, NS, group_rows, qc), jnp.bfloat16),
            pltpu.VMEM((NP - 1, 2 * NS, chunk, qc), jnp.bfloat16),
            pltpu.VMEM((NP - 1, 2 * NS, chunk, qc), jnp.bfloat16),
            pltpu.SemaphoreType.DMA((NP - 1, NS)),
            pltpu.SemaphoreType.DMA((NP - 1, NS)),
            pltpu.SemaphoreType.DMA((NP - 1, NS)),
            pltpu.SemaphoreType.DMA((NP - 1, NS)),
            pltpu.SemaphoreType.DMA((NP - 1, 2 * NS)),
            pltpu.SemaphoreType.DMA((NP - 1, 2 * NS)),
        ],
        compiler_params=pltpu.CompilerParams(collective_id=0),
    )(x, w_mat)


# device time: 22702 ns/iter; 1.0339x vs baseline; 1.0339x over previous
import contextlib
import os

import jax
import jax.numpy as jnp
from jax import lax
from jax.experimental import pallas as pl
from jax.experimental.pallas import tpu as pltpu


def _scope(name):
    if os.environ.get("KERNEL_SCOPES") == "1":
        return jax.named_scope(name)
    return contextlib.nullcontext()


N_DEV = 16
NP = 4
NS = 4


def kernel(x, w_mat):
    m, k_local = x.shape
    _, n = w_mat.shape
    chunk = m // N_DEV
    group_rows = NP * chunk
    hc = n // 2
    qc = hc // NS

    def body(x_ref, w_ref, out_ref, p_ref,
             cw_send, cw_recv, ccw_send, ccw_recv, p2_send, p2_recv,
             cw_ssem, cw_rsem, ccw_ssem, ccw_rsem, p2_ssem, p2_rsem):
        my = lax.axis_index("i")
        z = my // NP
        j = lax.rem(my, NP)
        plane_left = z * NP + lax.rem(j + NP - 1, NP)
        plane_right = z * NP + lax.rem(j + 1, NP)

        barrier_sem = pltpu.get_barrier_semaphore()
        for nbr in [plane_left, plane_right]:
            pl.semaphore_signal(
                barrier_sem, inc=1,
                device_id=(nbr,), device_id_type=pl.DeviceIdType.MESH,
            )

        with _scope("gemm"):
            xp = jnp.concatenate(
                [
                    x_ref[(NP * zz + g) * chunk:(NP * zz + g + 1) * chunk, :]
                    for g in range(NP)
                    for zz in range(NP)
                ]
            ).astype(jnp.bfloat16)
            wb = w_ref[...].astype(jnp.bfloat16)
            p_ref[...] = jnp.dot(
                xp, wb, preferred_element_type=jnp.float32
            ).astype(jnp.bfloat16)

        with _scope("barrier_wait"):
            pl.semaphore_wait(barrier_sem, 2)

        def pgroup(g, lo, width):
            return p_ref[pl.ds(g * group_rows, group_rows), lo:lo + width].astype(
                jnp.float32
            )

        ablate = os.environ.get("ABLATE", "")

        streams = []
        for k in range(NS):
            streams.append(
                (cw_send, cw_recv, cw_ssem, cw_rsem, k * qc, plane_right, +1, k)
            )
            streams.append(
                (ccw_send, ccw_recv, ccw_ssem, ccw_rsem, hc + k * qc, plane_left,
                 -1, k)
            )
        col_order = sorted(range(2 * NS), key=lambda i: streams[i][4])
        live = {}
        for s in range(NP - 1) if ablate not in ("p2", "none") else []:
            g_cw = lax.rem(j + 2 * NP - 1 - s, NP)
            g_ccw = lax.rem(j + 1 + s, NP)
            with _scope(f"hop{s}"):
                for idx, (sbuf, rbuf, ssem, rsem, lo, target, sgn, k) in enumerate(streams):
                    g = g_cw if sgn > 0 else g_ccw
                    acc = pgroup(g, lo, qc)
                    if s > 0:
                        live[(idx, s - 1)].wait_recv()
                        acc = acc + rbuf[s - 1, k].astype(jnp.float32)
                    sbuf[s, k] = acc.astype(jnp.bfloat16)
                    rdma = pltpu.make_async_remote_copy(
                        src_ref=sbuf.at[s, k],
                        dst_ref=rbuf.at[s, k],
                        send_sem=ssem.at[s, k],
                        recv_sem=rsem.at[s, k],
                        device_id=(target,),
                        device_id_type=pl.DeviceIdType.MESH,
                    )
                    rdma.start()
                    live[(idx, s)] = rdma

        def reduced_block(b, lo_k):
            sbuf, rbuf, ssem, rsem, lo, target, sgn, k = streams[lo_k]
            base = p_ref[
                pl.ds(j * group_rows + b * chunk, chunk), lo:lo + qc
            ].astype(jnp.float32)
            if ablate in ("p2", "none"):
                return base
            return (
                base
                + rbuf[NP - 2, k, pl.ds(b * chunk, chunk), :].astype(jnp.float32)
            )

        p2_rdmas = []
        own_pieces = []
        with _scope("p2_send"):
            for lo_k in range(2 * NS):
                if ablate not in ("p2", "none"):
                    live[(lo_k, NP - 2)].wait_recv()
                for r in (1, 2, 3) if ablate not in ("p1", "none") else []:
                    b = lax.rem(z + NP - r, NP)
                    p2_send[r - 1, lo_k] = reduced_block(b, lo_k).astype(
                        jnp.bfloat16
                    )
                    rdma = pltpu.make_async_remote_copy(
                        src_ref=p2_send.at[r - 1, lo_k],
                        dst_ref=p2_recv.at[r - 1, lo_k],
                        send_sem=p2_ssem.at[r - 1, lo_k],
                        recv_sem=p2_rsem.at[r - 1, lo_k],
                        device_id=(b * NP + j,),
                        device_id_type=pl.DeviceIdType.MESH,
                    )
                    rdma.start()
                    p2_rdmas.append(rdma)
                own_pieces.append(reduced_block(z, lo_k))

        with _scope("p2_recv"):
            for rdma in p2_rdmas:
                rdma.wait_recv()
        with _scope("out"):
            total = jnp.concatenate([own_pieces[i] for i in col_order], axis=1)
            if ablate not in ("p1", "none"):
                for r in (1, 2, 3):
                    total = total + jnp.concatenate(
                        [p2_recv[r - 1, lo_k] for lo_k in col_order], axis=1
                    ).astype(jnp.float32)
            out_ref[...] = jnp.maximum(total, 0.0)

            for rdma in live.values():
                rdma.wait_send()
            for rdma in p2_rdmas:
                rdma.wait_send()

    return pl.pallas_call(
        body,
        out_shape=jax.ShapeDtypeStruct((chunk, n), jnp.float32),
        in_specs=[
            pl.BlockSpec(memory_space=pltpu.VMEM),
            pl.BlockSpec(memory_space=pltpu.VMEM),
        ],
        out_specs=pl.BlockSpec(memory_space=pltpu.VMEM),
        scratch_shapes=[
            pltpu.VMEM((m, n), jnp.bfloat16),
            pltpu.VMEM((NP - 1, NS, group_rows, qc), jnp.bfloat16),
            pltpu.VMEM((NP - 1, NS, group_rows, qc), jnp.bfloat16),
            pltpu.VMEM((NP - 1, NS, group_rows, qc), jnp.bfloat16),
            pltpu.VMEM((NP - 1, NS, group_rows, qc), jnp.bfloat16),
            pltpu.VMEM((NP - 1, 2 * NS, chunk, qc), jnp.bfloat16),
            pltpu.VMEM((NP - 1, 2 * NS, chunk, qc), jnp.bfloat16),
            pltpu.SemaphoreType.DMA((NP - 1, NS)),
            pltpu.SemaphoreType.DMA((NP - 1, NS)),
            pltpu.SemaphoreType.DMA((NP - 1, NS)),
            pltpu.SemaphoreType.DMA((NP - 1, NS)),
            pltpu.SemaphoreType.DMA((NP - 1, 2 * NS)),
            pltpu.SemaphoreType.DMA((NP - 1, 2 * NS)),
        ],
        compiler_params=pltpu.CompilerParams(collective_id=0),
    )(x, w_mat)


# device time: 22622 ns/iter; 1.0375x vs baseline; 1.0035x over previous
import contextlib
import os

import jax
import jax.numpy as jnp
from jax import lax
from jax.experimental import pallas as pl
from jax.experimental.pallas import tpu as pltpu


def _scope(name):
    if os.environ.get("KERNEL_SCOPES") == "1":
        return jax.named_scope(name)
    return contextlib.nullcontext()


N_DEV = 16
NP = 4
NS = 4


def kernel(x, w_mat):
    m, k_local = x.shape
    _, n = w_mat.shape
    chunk = m // N_DEV
    group_rows = NP * chunk
    hc = n // 2
    qc = hc // NS

    def body(x_ref, w_ref, out_ref, p_ref,
             cw_send, cw_recv, ccw_send, ccw_recv, p2_send, p2_recv,
             cw_ssem, cw_rsem, ccw_ssem, ccw_rsem, p2_ssem, p2_rsem):
        my = lax.axis_index("i")
        z = my // NP
        j = lax.rem(my, NP)
        plane_left = z * NP + lax.rem(j + NP - 1, NP)
        plane_right = z * NP + lax.rem(j + 1, NP)

        barrier_sem = pltpu.get_barrier_semaphore()
        for nbr in [plane_left, plane_right]:
            pl.semaphore_signal(
                barrier_sem, inc=1,
                device_id=(nbr,), device_id_type=pl.DeviceIdType.MESH,
            )

        with _scope("gemm"):
            xp = jnp.concatenate(
                [
                    x_ref[(NP * zz + g) * chunk:(NP * zz + g + 1) * chunk, :]
                    for g in range(NP)
                    for zz in range(NP)
                ]
            ).astype(jnp.bfloat16)
            wb = w_ref[...].astype(jnp.bfloat16)
            p_ref[...] = jnp.dot(
                xp, wb, preferred_element_type=jnp.float32
            ).astype(jnp.bfloat16)

        with _scope("barrier_wait"):
            pl.semaphore_wait(barrier_sem, 2)

        def pgroup(g, lo, width):
            return p_ref[pl.ds(g * group_rows, group_rows), lo:lo + width].astype(
                jnp.float32
            )

        ablate = os.environ.get("ABLATE", "")

        streams = []
        for k in range(NS):
            streams.append(
                (cw_send, cw_recv, cw_ssem, cw_rsem, k * qc, plane_right, +1, k)
            )
            streams.append(
                (ccw_send, ccw_recv, ccw_ssem, ccw_rsem, hc + k * qc, plane_left,
                 -1, k)
            )
        col_order = sorted(range(2 * NS), key=lambda i: streams[i][4])
        live = {}
        for s in range(NP - 1) if ablate not in ("p2", "none") else []:
            g_cw = lax.rem(j + 2 * NP - 1 - s, NP)
            g_ccw = lax.rem(j + 1 + s, NP)
            with _scope(f"hop{s}"):
                for idx, (sbuf, rbuf, ssem, rsem, lo, target, sgn, k) in enumerate(streams):
                    g = g_cw if sgn > 0 else g_ccw
                    acc = pgroup(g, lo, qc)
                    if s > 0:
                        live[(idx, s - 1)].wait_recv()
                        acc = acc + rbuf[s - 1, k].astype(jnp.float32)
                    sbuf[s, k] = acc.astype(jnp.bfloat16)
                    rdma = pltpu.make_async_remote_copy(
                        src_ref=sbuf.at[s, k],
                        dst_ref=rbuf.at[s, k],
                        send_sem=ssem.at[s, k],
                        recv_sem=rsem.at[s, k],
                        device_id=(target,),
                        device_id_type=pl.DeviceIdType.MESH,
                    )
                    rdma.start()
                    live[(idx, s)] = rdma

        def reduced_block(b, lo_k):
            sbuf, rbuf, ssem, rsem, lo, target, sgn, k = streams[lo_k]
            base = p_ref[
                pl.ds(j * group_rows + b * chunk, chunk), lo:lo + qc
            ].astype(jnp.float32)
            if ablate in ("p2", "none"):
                return base
            return (
                base
                + rbuf[NP - 2, k, pl.ds(b * chunk, chunk), :].astype(jnp.float32)
            )

        p2_rdmas = []
        own_pieces = []
        with _scope("p2_send"):
            for lo_k in range(2 * NS):
                if ablate not in ("p2", "none"):
                    live[(lo_k, NP - 2)].wait_recv()
                for r in (1, 2, 3) if ablate not in ("p1", "none") else []:
                    b = lax.rem(z + NP - r, NP)
                    p2_send[r - 1, lo_k] = reduced_block(b, lo_k).astype(
                        jnp.bfloat16
                    )
                    rdma = pltpu.make_async_remote_copy(
                        src_ref=p2_send.at[r - 1, lo_k],
                        dst_ref=p2_recv.at[r - 1, lo_k],
                        send_sem=p2_ssem.at[r - 1, lo_k],
                        recv_sem=p2_rsem.at[r - 1, lo_k],
                        device_id=(b * NP + j,),
                        device_id_type=pl.DeviceIdType.MESH,
                    )
                    rdma.start()
                    p2_rdmas.append(rdma)
                own_pieces.append(reduced_block(z, lo_k))

        with _scope("p2_recv_out"):
            for lo_k in range(2 * NS):
                lo = streams[lo_k][4]
                piece = own_pieces[lo_k]
                if ablate not in ("p1", "none"):
                    for r in (1, 2, 3):
                        p2_rdmas[lo_k * 3 + (r - 1)].wait_recv()
                        piece = piece + p2_recv[r - 1, lo_k].astype(jnp.float32)
                out_ref[:, lo:lo + qc] = jnp.maximum(piece, 0.0)

            for rdma in live.values():
                rdma.wait_send()
            for rdma in p2_rdmas:
                rdma.wait_send()

    return pl.pallas_call(
        body,
        out_shape=jax.ShapeDtypeStruct((chunk, n), jnp.float32),
        in_specs=[
            pl.BlockSpec(memory_space=pltpu.VMEM),
            pl.BlockSpec(memory_space=pltpu.VMEM),
        ],
        out_specs=pl.BlockSpec(memory_space=pltpu.VMEM),
        scratch_shapes=[
            pltpu.VMEM((m, n), jnp.bfloat16),
            pltpu.VMEM((NP - 1, NS, group_rows, qc), jnp.bfloat16),
            pltpu.VMEM((NP - 1, NS, group_rows, qc), jnp.bfloat16),
            pltpu.VMEM((NP - 1, NS, group_rows, qc), jnp.bfloat16),
            pltpu.VMEM((NP - 1, NS, group_rows, qc), jnp.bfloat16),
            pltpu.VMEM((NP - 1, 2 * NS, chunk, qc), jnp.bfloat16),
            pltpu.VMEM((NP - 1, 2 * NS, chunk, qc), jnp.bfloat16),
            pltpu.SemaphoreType.DMA((NP - 1, NS)),
            pltpu.SemaphoreType.DMA((NP - 1, NS)),
            pltpu.SemaphoreType.DMA((NP - 1, NS)),
            pltpu.SemaphoreType.DMA((NP - 1, NS)),
            pltpu.SemaphoreType.DMA((NP - 1, 2 * NS)),
            pltpu.SemaphoreType.DMA((NP - 1, 2 * NS)),
        ],
        compiler_params=pltpu.CompilerParams(collective_id=0),
    )(x, w_mat)
